# baseline (device time: 9970 ns/iter reference)
import jax
import jax.numpy as jnp
from jax import lax
from jax.experimental import pallas as pl
from jax.experimental.pallas import tpu as pltpu

NCHUNK = 4


def kernel(x):
    m, n = x.shape
    ck = m // NCHUNK

    def body(x_ref, out_ref, x_vmem, send_buf, peer_buf, red_buf,
             in_sems, out_sems, p1_send, p1_recv, p2_send, p2_recv):
        my_x = lax.axis_index("x")
        my_y = lax.axis_index("y")
        x_peer = (1 - my_x, my_y)
        y_peer = (my_x, 1 - my_y)

        local_in = []
        for k in range(NCHUNK):
            sl = pl.ds(k * ck, ck)
            cp = pltpu.make_async_copy(x_ref.at[sl], x_vmem.at[sl],
                                       in_sems.at[k])
            cp.start()
            local_in.append(cp)

        barrier_sem = pltpu.get_barrier_semaphore()
        for nbr in (x_peer, y_peer):
            pl.semaphore_signal(
                barrier_sem, inc=1,
                device_id=nbr, device_id_type=pl.DeviceIdType.MESH,
            )
        pl.semaphore_wait(barrier_sem, 2)

        my_col = pl.ds(my_y * n, n)

        def p1_rdma(k):
            sl = pl.ds(k * ck, ck)
            return pltpu.make_async_remote_copy(
                src_ref=send_buf.at[sl],
                dst_ref=peer_buf.at[sl],
                send_sem=p1_send.at[k],
                recv_sem=p1_recv.at[k],
                device_id=x_peer,
                device_id_type=pl.DeviceIdType.MESH,
            )

        def p2_rdma(k):
            sl = pl.ds(k * ck, ck)
            return pltpu.make_async_remote_copy(
                src_ref=red_buf.at[sl],
                dst_ref=out_ref.at[sl, my_col],
                send_sem=p2_send.at[k],
                recv_sem=p2_recv.at[k],
                device_id=y_peer,
                device_id_type=pl.DeviceIdType.MESH,
            )

        p1 = [p1_rdma(k) for k in range(NCHUNK)]
        for k in range(NCHUNK):
            sl = pl.ds(k * ck, ck)
            local_in[k].wait()
            send_buf[sl] = x_vmem[sl].astype(jnp.bfloat16)
            p1[k].start()

        p2 = [p2_rdma(k) for k in range(NCHUNK)]
        local_out = []
        for k in range(NCHUNK):
            sl = pl.ds(k * ck, ck)
            p1[k].wait()
            red_buf[sl] = send_buf[sl] + peer_buf[sl]
            p2[k].start()
            cp = pltpu.make_async_copy(red_buf.at[sl],
                                       out_ref.at[sl, my_col],
                                       out_sems.at[k])
            cp.start()
            local_out.append(cp)

        for k in range(NCHUNK):
            local_out[k].wait()
        for k in range(NCHUNK):
            p2[k].wait()

    return pl.pallas_call(
        body,
        out_shape=jax.ShapeDtypeStruct((m, 2 * n), jnp.bfloat16),
        in_specs=[pl.BlockSpec(memory_space=pltpu.MemorySpace.HBM)],
        out_specs=pl.BlockSpec(memory_space=pltpu.MemorySpace.HBM),
        scratch_shapes=[
            pltpu.VMEM((m, n), jnp.float32),
            pltpu.VMEM((m, n), jnp.bfloat16),
            pltpu.VMEM((m, n), jnp.bfloat16),
            pltpu.VMEM((m, n), jnp.bfloat16),
            pltpu.SemaphoreType.DMA((NCHUNK,)),
            pltpu.SemaphoreType.DMA((NCHUNK,)),
            pltpu.SemaphoreType.DMA((NCHUNK,)),
            pltpu.SemaphoreType.DMA((NCHUNK,)),
            pltpu.SemaphoreType.DMA((NCHUNK,)),
            pltpu.SemaphoreType.DMA((NCHUNK,)),
        ],
        compiler_params=pltpu.CompilerParams(collective_id=0),
    )(x)


# device time: 9318 ns/iter; 1.0700x vs baseline; 1.0700x over previous
import jax
import jax.numpy as jnp
from jax import lax
from jax.experimental import pallas as pl
from jax.experimental.pallas import tpu as pltpu

NCHUNK = 8


def kernel(x):
    m, n = x.shape
    ck = m // NCHUNK

    def body(x_ref, out_ref, send_buf, peer_buf, p1_send, p1_recv,
             p2_send, p2_recv):
        my_x = lax.axis_index("x")
        my_y = lax.axis_index("y")
        x_peer = (1 - my_x, my_y)
        y_peer = (my_x, 1 - my_y)

        send_buf[...] = x_ref[...].astype(jnp.bfloat16)

        barrier_sem = pltpu.get_barrier_semaphore()
        for nbr in (x_peer, y_peer):
            pl.semaphore_signal(
                barrier_sem, inc=1,
                device_id=nbr, device_id_type=pl.DeviceIdType.MESH,
            )
        pl.semaphore_wait(barrier_sem, 2)

        my_col = pl.ds(my_y * n, n)

        def p1_rdma(k):
            sl = pl.ds(k * ck, ck)
            return pltpu.make_async_remote_copy(
                src_ref=send_buf.at[sl],
                dst_ref=peer_buf.at[sl],
                send_sem=p1_send.at[k],
                recv_sem=p1_recv.at[k],
                device_id=x_peer,
                device_id_type=pl.DeviceIdType.MESH,
            )

        def p2_rdma(k):
            sl = pl.ds(k * ck, ck)
            return pltpu.make_async_remote_copy(
                src_ref=out_ref.at[sl, my_col],
                dst_ref=out_ref.at[sl, my_col],
                send_sem=p2_send.at[k],
                recv_sem=p2_recv.at[k],
                device_id=y_peer,
                device_id_type=pl.DeviceIdType.MESH,
            )

        p1 = [p1_rdma(k) for k in range(NCHUNK)]
        for k in range(NCHUNK):
            p1[k].start()

        p2 = [p2_rdma(k) for k in range(NCHUNK)]
        for k in range(NCHUNK):
            sl = pl.ds(k * ck, ck)
            p1[k].wait()
            out_ref[sl, my_col] = send_buf[sl] + peer_buf[sl]
            p2[k].start()

        for k in range(NCHUNK):
            p2[k].wait()

    return pl.pallas_call(
        body,
        out_shape=jax.ShapeDtypeStruct((m, 2 * n), jnp.bfloat16),
        in_specs=[pl.BlockSpec(memory_space=pltpu.VMEM)],
        out_specs=pl.BlockSpec(memory_space=pltpu.VMEM),
        scratch_shapes=[
            pltpu.VMEM((m, n), jnp.bfloat16),
            pltpu.VMEM((m, n), jnp.bfloat16),
            pltpu.SemaphoreType.DMA((NCHUNK,)),
            pltpu.SemaphoreType.DMA((NCHUNK,)),
            pltpu.SemaphoreType.DMA((NCHUNK,)),
            pltpu.SemaphoreType.DMA((NCHUNK,)),
        ],
        compiler_params=pltpu.CompilerParams(collective_id=0),
    )(x)
